# baseline (device time: 45389 ns/iter reference)
import jax
import jax.numpy as jnp
from jax import lax
from jax.experimental import pallas as pl
from jax.experimental.pallas import tpu as pltpu

N_DEV = 4


def kernel(dy, W):
    m, _ = dy.shape
    n = W.shape[0]
    half = m // 4
    quar = m // 8

    def body(dy_ref, w_ref, out_ref, acc, wb, r1a, r1b, r2a, r2b,
             ssems, rsems):
        my = lax.axis_index("i")
        bit0 = my & 1
        bit1 = (my >> 1) & 1
        p1 = my ^ 1
        p3 = my ^ 3

        ka = bit0 ^ bit1
        ma = bit0
        kb = bit1
        mb = bit0

        barrier_sem = pltpu.get_barrier_semaphore()
        for nbr in (p1, p3):
            pl.semaphore_signal(
                barrier_sem, inc=1,
                device_id=(nbr,), device_id_type=pl.DeviceIdType.MESH,
            )
        pl.semaphore_wait(barrier_sem, 2)

        wb[...] = w_ref[...].astype(jnp.bfloat16)

        def gemm_rows(off):
            acc[pl.ds(off, half), :] = lax.dot_general(
                dy_ref[pl.ds(off, half), :].astype(jnp.bfloat16),
                wb[...],
                dimension_numbers=(((1,), (1,)), ((), ())),
                preferred_element_type=jnp.float32,
            ).astype(jnp.bfloat16)

        a_my_half = ka * half
        a_send_half = (1 - ka) * half
        a_my_q = ka * half + ma * quar
        a_send_q = ka * half + (1 - ma) * quar
        b0 = 2 * half
        b_my_half = b0 + kb * half
        b_send_half = b0 + (1 - kb) * half
        b_my_q = b0 + kb * half + mb * quar
        b_send_q = b0 + kb * half + (1 - mb) * quar

        def xchg(src, dst, idx, tgt):
            r = pltpu.make_async_remote_copy(
                src_ref=src, dst_ref=dst,
                send_sem=ssems.at[idx], recv_sem=rsems.at[idx],
                device_id=(tgt,), device_id_type=pl.DeviceIdType.MESH,
            )
            r.start()
            return r

        def reduce_rows(off, nrows, rbuf):
            cur = acc[pl.ds(off, nrows), :].astype(jnp.float32)
            acc[pl.ds(off, nrows), :] = (
                cur + rbuf[...].astype(jnp.float32)
            ).astype(jnp.bfloat16)

        def cast_out(off):
            out_ref[pl.ds(off, half), :] = acc[pl.ds(off, half), :].astype(
                jnp.float32)

        gemm_rows(a_send_half)
        p1a = xchg(acc.at[pl.ds(a_send_half, half)], r1a, 0, p1)
        gemm_rows(b_send_half)
        p1b = xchg(acc.at[pl.ds(b_send_half, half)], r1b, 1, p3)
        gemm_rows(a_my_half)
        gemm_rows(b_my_half)

        p1a.wait()
        reduce_rows(a_my_half, half, r1a)
        p2a = xchg(acc.at[pl.ds(a_send_q, quar)], r2a, 2, p3)
        p1b.wait()
        reduce_rows(b_my_half, half, r1b)
        p2b = xchg(acc.at[pl.ds(b_send_q, quar)], r2b, 3, p1)

        p2a.wait()
        reduce_rows(a_my_q, quar, r2a)
        p3a = xchg(acc.at[pl.ds(a_my_q, quar)], acc.at[pl.ds(a_my_q, quar)],
                   4, p3)
        p2b.wait()
        reduce_rows(b_my_q, quar, r2b)
        p3b = xchg(acc.at[pl.ds(b_my_q, quar)], acc.at[pl.ds(b_my_q, quar)],
                   5, p1)

        p3a.wait()
        p4a = xchg(acc.at[pl.ds(a_my_half, half)],
                   acc.at[pl.ds(a_my_half, half)], 6, p1)
        p3b.wait()
        p4b = xchg(acc.at[pl.ds(b_my_half, half)],
                   acc.at[pl.ds(b_my_half, half)], 7, p3)

        cast_out(a_my_half)
        cast_out(b_my_half)
        p4a.wait()
        cast_out(a_send_half)
        p4b.wait()
        cast_out(b_send_half)

    return pl.pallas_call(
        body,
        out_shape=jax.ShapeDtypeStruct((m, n), jnp.float32),
        in_specs=[
            pl.BlockSpec(memory_space=pltpu.VMEM),
            pl.BlockSpec(memory_space=pltpu.VMEM),
        ],
        out_specs=pl.BlockSpec(memory_space=pltpu.VMEM),
        scratch_shapes=[
            pltpu.VMEM((m, n), jnp.bfloat16),
            pltpu.VMEM((n, dy.shape[1]), jnp.bfloat16),
            pltpu.VMEM((half, n), jnp.bfloat16),
            pltpu.VMEM((half, n), jnp.bfloat16),
            pltpu.VMEM((quar, n), jnp.bfloat16),
            pltpu.VMEM((quar, n), jnp.bfloat16),
            pltpu.SemaphoreType.DMA((8,)),
            pltpu.SemaphoreType.DMA((8,)),
        ],
        compiler_params=pltpu.CompilerParams(collective_id=0),
    )(dy, W)
